# Initial kernel scaffold; baseline (speedup 1.0000x reference)
import jax
import jax.numpy as jnp
from jax.experimental import pallas as pl


def kernel(token_ids, embedding_weights):
    raise NotImplementedError("write your pallas kernel here")


# speedup vs baseline: 1.6854x; 1.6854x over previous
import functools

import jax
import jax.numpy as jnp
from jax import lax
from jax.experimental import pallas as pl
from jax.experimental.pallas import tpu as pltpu
from jax.experimental.pallas import tpu_sc as plsc

BATCH = 16384
HIST = 50
DIM = 64
TOTAL = BATCH * HIST

NUM_CORES = 2
NUM_SUBCORES = 16
NUM_WORKERS = NUM_CORES * NUM_SUBCORES
ROWS_PER_WORKER = TOTAL // NUM_WORKERS
CHUNK = 128
NUM_CHUNKS = ROWS_PER_WORKER // CHUNK


def _make_kernel():
    mesh = plsc.VectorSubcoreMesh(
        core_axis_name="c", subcore_axis_name="s", num_cores=NUM_CORES
    )

    @functools.partial(
        pl.kernel,
        mesh=mesh,
        out_type=jax.ShapeDtypeStruct((TOTAL, DIM), jnp.float32),
        scratch_types=[
            pltpu.VMEM((ROWS_PER_WORKER,), jnp.int32),
            pltpu.VMEM((CHUNK, DIM), jnp.float32),
            pltpu.SemaphoreType.DMA,
        ],
        compiler_params=pltpu.CompilerParams(use_tc_tiling_on_sc=False),
    )
    def lookup(ids_hbm, table_hbm, out_hbm, idx_v, rows_v, sem):
        wid = lax.axis_index("s") * NUM_CORES + lax.axis_index("c")
        base = wid * ROWS_PER_WORKER
        pltpu.sync_copy(ids_hbm.at[pl.ds(base, ROWS_PER_WORKER)], idx_v)

        def step(c, carry):
            off = c * CHUNK
            idx_slice = idx_v.at[pl.ds(off, CHUNK)]
            pltpu.async_copy(table_hbm.at[idx_slice], rows_v, sem).wait()
            pltpu.sync_copy(rows_v, out_hbm.at[pl.ds(base + off, CHUNK)])
            return carry

        lax.fori_loop(0, NUM_CHUNKS, step, 0)

    return lookup


_lookup = _make_kernel()


@jax.jit
def kernel(token_ids, embedding_weights):
    flat_ids = token_ids.reshape(TOTAL)
    out = _lookup(flat_ids, embedding_weights)
    return out.reshape(BATCH, HIST, DIM)


# speedup vs baseline: 1.8763x; 1.1133x over previous
import functools

import jax
import jax.numpy as jnp
from jax import lax
from jax.experimental import pallas as pl
from jax.experimental.pallas import tpu as pltpu
from jax.experimental.pallas import tpu_sc as plsc

BATCH = 16384
HIST = 50
DIM = 64
TOTAL = BATCH * HIST

NUM_CORES = 2
NUM_SUBCORES = 16
NUM_WORKERS = NUM_CORES * NUM_SUBCORES
ROWS_PER_WORKER = TOTAL // NUM_WORKERS
CHUNK = 128
NUM_CHUNKS = ROWS_PER_WORKER // CHUNK
NBUF = 8
LAG = 6


def _make_kernel():
    mesh = plsc.VectorSubcoreMesh(
        core_axis_name="c", subcore_axis_name="s", num_cores=NUM_CORES
    )

    @functools.partial(
        pl.kernel,
        mesh=mesh,
        out_type=jax.ShapeDtypeStruct((TOTAL, DIM), jnp.float32),
        scratch_types=(
            [pltpu.VMEM((ROWS_PER_WORKER,), jnp.int32)]
            + [pltpu.VMEM((CHUNK, DIM), jnp.float32)] * NBUF
            + [pltpu.SemaphoreType.DMA] * (2 * NBUF)
        ),
        compiler_params=pltpu.CompilerParams(use_tc_tiling_on_sc=False),
    )
    def lookup(ids_hbm, table_hbm, out_hbm, idx_v, *rest):
        bufs = rest[:NBUF]
        gsem = rest[NBUF : 2 * NBUF]
        ssem = rest[2 * NBUF :]

        wid = lax.axis_index("s") * NUM_CORES + lax.axis_index("c")
        base = wid * ROWS_PER_WORKER
        pltpu.sync_copy(ids_hbm.at[pl.ds(base, ROWS_PER_WORKER)], idx_v)

        def idx_slice(t):
            return idx_v.at[pl.ds(t * CHUNK, CHUNK)]

        def start_gather(t, b):
            pltpu.async_copy(table_hbm.at[idx_slice(t)], bufs[b], gsem[b])

        def wait_gather(t, b):
            pltpu.make_async_copy(
                table_hbm.at[idx_slice(t)], bufs[b], gsem[b]
            ).wait()

        def start_store(t, b):
            pltpu.async_copy(
                bufs[b], out_hbm.at[pl.ds(base + t * CHUNK, CHUNK)], ssem[b]
            )

        def drain_store(b):
            pltpu.make_async_copy(
                out_hbm.at[pl.ds(base, CHUNK)], bufs[b], ssem[b]
            ).wait()

        for t in range(LAG):
            start_gather(t, t)
        for t in range(LAG, NBUF):
            wait_gather(t - LAG, t - LAG)
            start_store(t - LAG, t - LAG)
            start_gather(t, t)

        def group(g, carry):
            for b in range(NBUF):
                t = g * NBUF + b
                drain_store(b)
                start_gather(t, b)
                b2 = (b + NBUF - LAG) % NBUF
                wait_gather(t - LAG, b2)
                start_store(t - LAG, b2)
            return carry

        lax.fori_loop(1, NUM_CHUNKS // NBUF, group, 0)

        for t in range(NUM_CHUNKS, NUM_CHUNKS + LAG):
            t2 = t - LAG
            wait_gather(t2, t2 % NBUF)
            start_store(t2, t2 % NBUF)
        for b in range(NBUF):
            drain_store(b)

    return lookup


_lookup = _make_kernel()


@jax.jit
def kernel(token_ids, embedding_weights):
    flat_ids = token_ids.reshape(TOTAL)
    out = _lookup(flat_ids, embedding_weights)
    return out.reshape(BATCH, HIST, DIM)
